# baseline (speedup 1.0000x reference)
import functools

import jax
import jax.numpy as jnp
from jax import lax
from jax.experimental import pallas as pl
from jax.experimental.pallas import tpu as pltpu
from jax.experimental.pallas import tpu_sc as plsc

N, E, B, H = 10000, 320000, 256, 128

BK = 2560
NBK_ALL = E // BK
NBK_SC = 65
E_SC = NBK_SC * BK
NBK = NBK_ALL - NBK_SC

LN = 16
NC, NS = 2, 16
NW = NC * NS
EPW = E_SC // NW
CHUNK = 80
NCH = EPW // CHUNK
NBUF = 4

_mesh = plsc.VectorSubcoreMesh(core_axis_name="c", subcore_axis_name="s")


@functools.partial(
    pl.kernel,
    out_type=(jax.ShapeDtypeStruct((NC, B, H), jnp.float32),
              jax.ShapeDtypeStruct((NW, B * LN), jnp.float32)),
    mesh=_mesh,
    scratch_types=[
        pltpu.VMEM((N,), jnp.int32),
        pltpu.VMEM((EPW,), jnp.int32),
        pltpu.VMEM((NBUF, CHUNK, H), jnp.float32),
        pltpu.VMEM((NBUF, CHUNK), jnp.int32),
        pltpu.VMEM_SHARED((B, H), jnp.float32),
        pltpu.VMEM((B * LN,), jnp.float32),
        [pltpu.SemaphoreType.DMA] * NBUF,
        [pltpu.SemaphoreType.DMA] * NBUF,
        pltpu.SemaphoreType.DMA,
    ],
    compiler_params=pltpu.CompilerParams(needs_layout_passes=False),
)
def _sc_edge_pool(row_hbm, batch_hbm, edge_hbm, zsum_hbm,
                  sums_hbm, cnts_hbm,
                  batch_v, row_v, ebuf, idx_v, acc_v, cnt_v,
                  dsem, ssem, gsem):
    sid = lax.axis_index("s")
    cid = lax.axis_index("c")
    wid = sid * NC + cid
    base = wid * EPW

    pltpu.async_copy(batch_hbm, batch_v, gsem)
    pltpu.async_copy(row_hbm.at[pl.ds(base, EPW)], row_v, gsem)

    @pl.when(sid == 0)
    def _zero_shared():
        pltpu.sync_copy(zsum_hbm, acc_v)

    zeros16 = jnp.zeros((LN,), jnp.float32)
    ones16 = jnp.ones((LN,), jnp.float32)
    lane_iota = lax.iota(jnp.int32, LN)

    def _zero_cnt(i, carry):
        for k in range(16):
            cnt_v[pl.ds(i * 256 + k * LN, LN)] = zeros16
        return carry
    lax.fori_loop(0, (B * LN) // 256, _zero_cnt, 0)

    pltpu.make_async_copy(batch_hbm, batch_v, gsem).wait()
    pltpu.make_async_copy(row_hbm.at[pl.ds(base, EPW)], row_v, gsem).wait()
    plsc.subcore_barrier()

    def _chunk_src(c):
        return edge_hbm.at[pl.ds(base + c * CHUNK, CHUNK), :]

    def _fill_idx(c, s):
        for k in range(CHUNK // LN):
            r16 = row_v[pl.ds(c * CHUNK + k * LN, LN)]
            idx_v[s, pl.ds(k * LN, LN)] = plsc.load_gather(batch_v, [r16])

    def _scatter_drain(s):
        pltpu.make_async_copy(ebuf.at[s], acc_v.at[idx_v.at[s]],
                              ssem[s]).wait()

    pltpu.async_copy(_chunk_src(0), ebuf.at[0], dsem[0])
    pltpu.async_copy(_chunk_src(1), ebuf.at[1], dsem[1])

    def _turn(cc, carry):
        for s in range(NBUF):
            c = cc * NBUF + s

            @pl.when(c < NCH)
            def _process():
                pltpu.make_async_copy(_chunk_src(0), ebuf.at[s],
                                      dsem[s]).wait()
                _fill_idx(c, s)
                pltpu.async_copy(ebuf.at[s], acc_v.at[idx_v.at[s]], ssem[s],
                                 add=True)
                for k in range(CHUNK // LN):
                    sv = idx_v[s, pl.ds(k * LN, LN)]
                    tgt = sv * LN + lane_iota
                    plsc.addupdate_scatter(cnt_v, [tgt], ones16)

            sp = (s + 2) % NBUF

            @pl.when(c + 2 < NCH)
            def _prefetch():
                @pl.when(c >= 2)
                def _drain_prev():
                    _scatter_drain(sp)
                pltpu.async_copy(_chunk_src(c + 2), ebuf.at[sp], dsem[sp])
        return carry
    lax.fori_loop(0, (NCH + NBUF - 1) // NBUF, _turn, 0)

    for sf in range(NBUF):
        _scatter_drain(sf)
    plsc.subcore_barrier()

    @pl.when(sid == 0)
    def _out_sums():
        pltpu.sync_copy(acc_v, sums_hbm.at[cid])
    pltpu.sync_copy(cnt_v, cnts_hbm.at[wid])


def _tc_x_body(x_ref, batch_ref, row_ref, e_ref, xm_ref, es_ref, ec_ref,
               starts_scr, hist_scr, acc_scr, cnt_scr):
    i = pl.program_id(0)

    @pl.when(i == 0)
    def _init():
        b_iota = jax.lax.broadcasted_iota(jnp.int32, (B, N), 0)
        hist_col = jnp.sum(jnp.equal(batch_ref[...], b_iota).astype(jnp.float32),
                           axis=1, keepdims=True)
        tri = (jax.lax.broadcasted_iota(jnp.int32, (B, B), 0)
               > jax.lax.broadcasted_iota(jnp.int32, (B, B), 1)).astype(jnp.float32)
        starts_col = jnp.dot(tri, hist_col, preferred_element_type=jnp.float32)
        starts_scr[...] = jnp.broadcast_to(starts_col.astype(jnp.int32), (B, H))
        hist_scr[...] = jnp.broadcast_to(hist_col.astype(jnp.int32), (B, H))
        acc_scr[...] = jnp.zeros((B, H), jnp.float32)
        cnt_scr[...] = jnp.zeros((B, H), jnp.float32)

    s_col = starts_scr[:, 0:1]
    h_col = hist_scr[:, 0:1]
    e_col = s_col + h_col

    row2 = row_ref[...].reshape(1, BK)
    mask = (row2 >= s_col) & (row2 < e_col)
    onehot = mask.astype(jnp.bfloat16)
    eblk = e_ref[...].astype(jnp.bfloat16)
    acc_scr[...] += jnp.dot(onehot, eblk, preferred_element_type=jnp.float32)
    cnt_blk = jnp.sum(mask.astype(jnp.float32), axis=1, keepdims=True)
    cnt_scr[...] += jnp.broadcast_to(cnt_blk, (B, H))

    @pl.when(i == NBK - 1)
    def _finish():
        n_iota = jax.lax.broadcasted_iota(jnp.int32, (B, N), 1)
        maskx = ((n_iota >= s_col) & (n_iota < e_col)).astype(jnp.float32)
        sum_x = jnp.dot(maskx, x_ref[...], preferred_element_type=jnp.float32)
        hist_f = h_col.astype(jnp.float32)
        xm_ref[...] = sum_x / jnp.maximum(hist_f, 1.0)
        es_ref[...] = acc_scr[...]
        ec_ref[...] = cnt_scr[...]


def _tc_x(x, batch2, row3, edge_attr):
    cmap = lambda i: (0, 0)
    return pl.pallas_call(
        _tc_x_body,
        grid=(NBK,),
        in_specs=[
            pl.BlockSpec((N, H), cmap),
            pl.BlockSpec((1, N), cmap),
            pl.BlockSpec((1, 1, BK), lambda i: (i + NBK_SC, 0, 0)),
            pl.BlockSpec((BK, H), lambda i: (i + NBK_SC, 0)),
        ],
        out_specs=[pl.BlockSpec((B, H), cmap)] * 3,
        out_shape=[jax.ShapeDtypeStruct((B, H), jnp.float32)] * 3,
        scratch_shapes=[
            pltpu.VMEM((B, H), jnp.int32),
            pltpu.VMEM((B, H), jnp.int32),
            pltpu.VMEM((B, H), jnp.float32),
            pltpu.VMEM((B, H), jnp.float32),
        ],
        compiler_params=pltpu.CompilerParams(
            dimension_semantics=("arbitrary",),
        ),
    )(x, batch2, row3, edge_attr)


def _tc_combine_body(ps_ref, pc_ref, es_ref, ec_ref, xm_ref, u_ref, w1_ref,
                     b1_ref, w2_ref, b2_ref, out_ref):
    dn = (((1,), (1,)), ((), ()))
    e_sum = jnp.sum(ps_ref[...], axis=0) + es_ref[...]
    cnt_col = (jnp.sum(jnp.sum(pc_ref[...], axis=0), axis=1,
                       keepdims=True) + ec_ref[:, 0:1])
    e_mean = e_sum / jnp.maximum(cnt_col, 1.0)
    cat = jnp.concatenate([u_ref[...], xm_ref[...], e_mean], axis=1)
    h1 = jax.lax.dot_general(cat, w1_ref[...], dn,
                             preferred_element_type=jnp.float32) + b1_ref[...]
    h1 = jnp.maximum(h1, 0.0)
    out_ref[...] = jax.lax.dot_general(h1, w2_ref[...], dn,
                                       preferred_element_type=jnp.float32) + b2_ref[...]


def _tc_combine(part_sums, part_cnts, e_tc_sum, e_tc_cnt, x_mean, u, W1, b1r,
                W2, b2r):
    return pl.pallas_call(
        _tc_combine_body,
        grid=(1,),
        in_specs=[
            pl.BlockSpec((NC, B, H), lambda i: (0, 0, 0)),
            pl.BlockSpec((NW, B, LN), lambda i: (0, 0, 0)),
            pl.BlockSpec((B, H), lambda i: (0, 0)),
            pl.BlockSpec((B, H), lambda i: (0, 0)),
            pl.BlockSpec((B, H), lambda i: (0, 0)),
            pl.BlockSpec((B, H), lambda i: (0, 0)),
            pl.BlockSpec((H, 3 * H), lambda i: (0, 0)),
            pl.BlockSpec((1, H), lambda i: (0, 0)),
            pl.BlockSpec((H, H), lambda i: (0, 0)),
            pl.BlockSpec((1, H), lambda i: (0, 0)),
        ],
        out_specs=pl.BlockSpec((B, H), lambda i: (0, 0)),
        out_shape=jax.ShapeDtypeStruct((B, H), jnp.float32),
        compiler_params=pltpu.CompilerParams(
            dimension_semantics=("arbitrary",),
        ),
    )(part_sums, part_cnts, e_tc_sum, e_tc_cnt, x_mean, u, W1, b1r, W2, b2r)


def kernel(x, edge_index, edge_attr, u, batch, W1, b1, W2, b2):
    row = edge_index[0]
    zsum = jnp.zeros((B, H), jnp.float32)
    part_sums, part_cnts = _sc_edge_pool(row, batch, edge_attr, zsum)
    x_mean, e_tc_sum, e_tc_cnt = _tc_x(x, batch.reshape(1, N),
                                       row.reshape(NBK_ALL, 1, BK), edge_attr)
    return _tc_combine(part_sums, part_cnts.reshape(NW, B, LN), e_tc_sum,
                       e_tc_cnt, x_mean, u, W1, b1.reshape(1, H), W2,
                       b2.reshape(1, H))


# speedup vs baseline: 1.0616x; 1.0616x over previous
import functools

import jax
import jax.numpy as jnp
from jax import lax
from jax.experimental import pallas as pl
from jax.experimental.pallas import tpu as pltpu
from jax.experimental.pallas import tpu_sc as plsc

N, E, B, H = 10000, 320000, 256, 128

BK = 2560
NBK_ALL = E // BK
NBK_SC = 72
E_SC = NBK_SC * BK
NBK = NBK_ALL - NBK_SC

LN = 16
NC, NS = 2, 16
NW = NC * NS
EPW = E_SC // NW
CHUNK = 80
NCH = EPW // CHUNK
NBUF = 4

_mesh = plsc.VectorSubcoreMesh(core_axis_name="c", subcore_axis_name="s")


@functools.partial(
    pl.kernel,
    out_type=(jax.ShapeDtypeStruct((NC, B, H), jnp.float32),
              jax.ShapeDtypeStruct((NW, B * LN), jnp.float32)),
    mesh=_mesh,
    scratch_types=[
        pltpu.VMEM((N,), jnp.int32),
        pltpu.VMEM((EPW,), jnp.int32),
        pltpu.VMEM((NBUF, CHUNK, H), jnp.float32),
        pltpu.VMEM((NBUF, CHUNK), jnp.int32),
        pltpu.VMEM_SHARED((B, H), jnp.float32),
        pltpu.VMEM((B * LN,), jnp.float32),
        [pltpu.SemaphoreType.DMA] * NBUF,
        [pltpu.SemaphoreType.DMA] * NBUF,
        pltpu.SemaphoreType.DMA,
    ],
    compiler_params=pltpu.CompilerParams(needs_layout_passes=False),
)
def _sc_edge_pool(row_hbm, batch_hbm, edge_hbm, zsum_hbm,
                  sums_hbm, cnts_hbm,
                  batch_v, row_v, ebuf, idx_v, acc_v, cnt_v,
                  dsem, ssem, gsem):
    sid = lax.axis_index("s")
    cid = lax.axis_index("c")
    wid = sid * NC + cid
    base = wid * EPW

    pltpu.async_copy(batch_hbm, batch_v, gsem)
    pltpu.async_copy(row_hbm.at[pl.ds(base, EPW)], row_v, gsem)

    @pl.when(sid == 0)
    def _zero_shared():
        pltpu.sync_copy(zsum_hbm, acc_v)

    zeros16 = jnp.zeros((LN,), jnp.float32)
    ones16 = jnp.ones((LN,), jnp.float32)
    lane_iota = lax.iota(jnp.int32, LN)

    def _zero_cnt(i, carry):
        for k in range(16):
            cnt_v[pl.ds(i * 256 + k * LN, LN)] = zeros16
        return carry
    lax.fori_loop(0, (B * LN) // 256, _zero_cnt, 0)

    pltpu.make_async_copy(batch_hbm, batch_v, gsem).wait()
    pltpu.make_async_copy(row_hbm.at[pl.ds(base, EPW)], row_v, gsem).wait()
    plsc.subcore_barrier()

    def _chunk_src(c):
        return edge_hbm.at[pl.ds(base + c * CHUNK, CHUNK), :]

    def _fill_idx(c, s):
        for k in range(CHUNK // LN):
            r16 = row_v[pl.ds(c * CHUNK + k * LN, LN)]
            idx_v[s, pl.ds(k * LN, LN)] = plsc.load_gather(batch_v, [r16])

    def _scatter_drain(s):
        pltpu.make_async_copy(ebuf.at[s], acc_v.at[idx_v.at[s]],
                              ssem[s]).wait()

    pltpu.async_copy(_chunk_src(0), ebuf.at[0], dsem[0])
    pltpu.async_copy(_chunk_src(1), ebuf.at[1], dsem[1])

    def _turn(cc, carry):
        for s in range(NBUF):
            c = cc * NBUF + s

            @pl.when(c < NCH)
            def _process():
                pltpu.make_async_copy(_chunk_src(0), ebuf.at[s],
                                      dsem[s]).wait()
                _fill_idx(c, s)
                pltpu.async_copy(ebuf.at[s], acc_v.at[idx_v.at[s]], ssem[s],
                                 add=True)
                for k in range(CHUNK // LN):
                    sv = idx_v[s, pl.ds(k * LN, LN)]
                    tgt = sv * LN + lane_iota
                    plsc.addupdate_scatter(cnt_v, [tgt], ones16)

            sp = (s + 2) % NBUF

            @pl.when(c + 2 < NCH)
            def _prefetch():
                @pl.when(c >= 2)
                def _drain_prev():
                    _scatter_drain(sp)
                pltpu.async_copy(_chunk_src(c + 2), ebuf.at[sp], dsem[sp])
        return carry
    lax.fori_loop(0, (NCH + NBUF - 1) // NBUF, _turn, 0)

    for sf in range(NBUF):
        _scatter_drain(sf)
    plsc.subcore_barrier()

    @pl.when(sid == 0)
    def _out_sums():
        pltpu.sync_copy(acc_v, sums_hbm.at[cid])
    pltpu.sync_copy(cnt_v, cnts_hbm.at[wid])


def _tc_x_body(x_ref, batch_ref, row_ref, e_ref, xm_ref, es_ref, ec_ref,
               starts_scr, hist_scr, acc_scr, cnt_scr):
    i = pl.program_id(0)

    @pl.when(i == 0)
    def _init():
        b_iota = jax.lax.broadcasted_iota(jnp.int32, (B, N), 0)
        hist_col = jnp.sum(jnp.equal(batch_ref[...], b_iota).astype(jnp.float32),
                           axis=1, keepdims=True)
        tri = (jax.lax.broadcasted_iota(jnp.int32, (B, B), 0)
               > jax.lax.broadcasted_iota(jnp.int32, (B, B), 1)).astype(jnp.float32)
        starts_col = jnp.dot(tri, hist_col, preferred_element_type=jnp.float32)
        starts_scr[...] = jnp.broadcast_to(starts_col.astype(jnp.int32), (B, H))
        hist_scr[...] = jnp.broadcast_to(hist_col.astype(jnp.int32), (B, H))
        acc_scr[...] = jnp.zeros((B, H), jnp.float32)
        cnt_scr[...] = jnp.zeros((B, H), jnp.float32)

    s_col = starts_scr[:, 0:1]
    h_col = hist_scr[:, 0:1]
    e_col = s_col + h_col

    row2 = row_ref[...].reshape(1, BK)
    mask = (row2 >= s_col) & (row2 < e_col)
    onehot = mask.astype(jnp.bfloat16)
    eblk = e_ref[...].astype(jnp.bfloat16)
    acc_scr[...] += jnp.dot(onehot, eblk, preferred_element_type=jnp.float32)
    cnt_blk = jnp.sum(mask.astype(jnp.float32), axis=1, keepdims=True)
    cnt_scr[...] += jnp.broadcast_to(cnt_blk, (B, H))

    @pl.when(i == NBK - 1)
    def _finish():
        n_iota = jax.lax.broadcasted_iota(jnp.int32, (B, N), 1)
        maskx = ((n_iota >= s_col) & (n_iota < e_col)).astype(jnp.float32)
        sum_x = jnp.dot(maskx, x_ref[...], preferred_element_type=jnp.float32)
        hist_f = h_col.astype(jnp.float32)
        xm_ref[...] = sum_x / jnp.maximum(hist_f, 1.0)
        es_ref[...] = acc_scr[...]
        ec_ref[...] = cnt_scr[...]


def _tc_x(x, batch2, row3, edge_attr):
    cmap = lambda i: (0, 0)
    return pl.pallas_call(
        _tc_x_body,
        grid=(NBK,),
        in_specs=[
            pl.BlockSpec((N, H), cmap),
            pl.BlockSpec((1, N), cmap),
            pl.BlockSpec((1, 1, BK), lambda i: (i + NBK_SC, 0, 0)),
            pl.BlockSpec((BK, H), lambda i: (i + NBK_SC, 0)),
        ],
        out_specs=[pl.BlockSpec((B, H), cmap)] * 3,
        out_shape=[jax.ShapeDtypeStruct((B, H), jnp.float32)] * 3,
        scratch_shapes=[
            pltpu.VMEM((B, H), jnp.int32),
            pltpu.VMEM((B, H), jnp.int32),
            pltpu.VMEM((B, H), jnp.float32),
            pltpu.VMEM((B, H), jnp.float32),
        ],
        compiler_params=pltpu.CompilerParams(
            dimension_semantics=("arbitrary",),
        ),
    )(x, batch2, row3, edge_attr)


def _tc_combine_body(ps_ref, pc_ref, es_ref, ec_ref, xm_ref, u_ref, w1_ref,
                     b1_ref, w2_ref, b2_ref, out_ref):
    dn = (((1,), (1,)), ((), ()))
    e_sum = jnp.sum(ps_ref[...], axis=0) + es_ref[...]
    cnt_col = (jnp.sum(jnp.sum(pc_ref[...], axis=0), axis=1,
                       keepdims=True) + ec_ref[:, 0:1])
    e_mean = e_sum / jnp.maximum(cnt_col, 1.0)
    cat = jnp.concatenate([u_ref[...], xm_ref[...], e_mean], axis=1)
    h1 = jax.lax.dot_general(cat, w1_ref[...], dn,
                             preferred_element_type=jnp.float32) + b1_ref[...]
    h1 = jnp.maximum(h1, 0.0)
    out_ref[...] = jax.lax.dot_general(h1, w2_ref[...], dn,
                                       preferred_element_type=jnp.float32) + b2_ref[...]


def _tc_combine(part_sums, part_cnts, e_tc_sum, e_tc_cnt, x_mean, u, W1, b1r,
                W2, b2r):
    return pl.pallas_call(
        _tc_combine_body,
        grid=(1,),
        in_specs=[
            pl.BlockSpec((NC, B, H), lambda i: (0, 0, 0)),
            pl.BlockSpec((NW, B, LN), lambda i: (0, 0, 0)),
            pl.BlockSpec((B, H), lambda i: (0, 0)),
            pl.BlockSpec((B, H), lambda i: (0, 0)),
            pl.BlockSpec((B, H), lambda i: (0, 0)),
            pl.BlockSpec((B, H), lambda i: (0, 0)),
            pl.BlockSpec((H, 3 * H), lambda i: (0, 0)),
            pl.BlockSpec((1, H), lambda i: (0, 0)),
            pl.BlockSpec((H, H), lambda i: (0, 0)),
            pl.BlockSpec((1, H), lambda i: (0, 0)),
        ],
        out_specs=pl.BlockSpec((B, H), lambda i: (0, 0)),
        out_shape=jax.ShapeDtypeStruct((B, H), jnp.float32),
        compiler_params=pltpu.CompilerParams(
            dimension_semantics=("arbitrary",),
        ),
    )(part_sums, part_cnts, e_tc_sum, e_tc_cnt, x_mean, u, W1, b1r, W2, b2r)


def kernel(x, edge_index, edge_attr, u, batch, W1, b1, W2, b2):
    row = edge_index[0]
    zsum = jnp.zeros((B, H), jnp.float32)
    part_sums, part_cnts = _sc_edge_pool(row, batch, edge_attr, zsum)
    x_mean, e_tc_sum, e_tc_cnt = _tc_x(x, batch.reshape(1, N),
                                       row.reshape(NBK_ALL, 1, BK), edge_attr)
    return _tc_combine(part_sums, part_cnts.reshape(NW, B, LN), e_tc_sum,
                       e_tc_cnt, x_mean, u, W1, b1.reshape(1, H), W2,
                       b2.reshape(1, H))


# speedup vs baseline: 1.0918x; 1.0285x over previous
import functools

import jax
import jax.numpy as jnp
from jax import lax
from jax.experimental import pallas as pl
from jax.experimental.pallas import tpu as pltpu
from jax.experimental.pallas import tpu_sc as plsc

N, E, B, H = 10000, 320000, 256, 128

BK = 2560
NBK_ALL = E // BK
NBK_SC = 75
E_SC = NBK_SC * BK
NBK = NBK_ALL - NBK_SC

LN = 16
NC, NS = 2, 16
NW = NC * NS
EPW = E_SC // NW
CHUNK = 80
NCH = EPW // CHUNK
NBUF = 4

_mesh = plsc.VectorSubcoreMesh(core_axis_name="c", subcore_axis_name="s")


@functools.partial(
    pl.kernel,
    out_type=(jax.ShapeDtypeStruct((NC, B, H), jnp.float32),
              jax.ShapeDtypeStruct((NW, B * LN), jnp.float32)),
    mesh=_mesh,
    scratch_types=[
        pltpu.VMEM((N,), jnp.int32),
        pltpu.VMEM((EPW,), jnp.int32),
        pltpu.VMEM((NBUF, CHUNK, H), jnp.float32),
        pltpu.VMEM((NBUF, CHUNK), jnp.int32),
        pltpu.VMEM_SHARED((B, H), jnp.float32),
        pltpu.VMEM((B * LN,), jnp.float32),
        [pltpu.SemaphoreType.DMA] * NBUF,
        [pltpu.SemaphoreType.DMA] * NBUF,
        pltpu.SemaphoreType.DMA,
    ],
    compiler_params=pltpu.CompilerParams(needs_layout_passes=False),
)
def _sc_edge_pool(row_hbm, batch_hbm, edge_hbm, zsum_hbm,
                  sums_hbm, cnts_hbm,
                  batch_v, row_v, ebuf, idx_v, acc_v, cnt_v,
                  dsem, ssem, gsem):
    sid = lax.axis_index("s")
    cid = lax.axis_index("c")
    wid = sid * NC + cid
    base = wid * EPW

    pltpu.async_copy(batch_hbm, batch_v, gsem)
    pltpu.async_copy(row_hbm.at[pl.ds(base, EPW)], row_v, gsem)

    @pl.when(sid == 0)
    def _zero_shared():
        pltpu.sync_copy(zsum_hbm, acc_v)

    zeros16 = jnp.zeros((LN,), jnp.float32)
    ones16 = jnp.ones((LN,), jnp.float32)
    lane_iota = lax.iota(jnp.int32, LN)

    def _zero_cnt(i, carry):
        for k in range(16):
            cnt_v[pl.ds(i * 256 + k * LN, LN)] = zeros16
        return carry
    lax.fori_loop(0, (B * LN) // 256, _zero_cnt, 0)

    pltpu.make_async_copy(batch_hbm, batch_v, gsem).wait()
    pltpu.make_async_copy(row_hbm.at[pl.ds(base, EPW)], row_v, gsem).wait()
    plsc.subcore_barrier()

    def _chunk_src(c):
        return edge_hbm.at[pl.ds(base + c * CHUNK, CHUNK), :]

    def _fill_idx(c, s):
        for k in range(CHUNK // LN):
            r16 = row_v[pl.ds(c * CHUNK + k * LN, LN)]
            idx_v[s, pl.ds(k * LN, LN)] = plsc.load_gather(batch_v, [r16])

    def _scatter_drain(s):
        pltpu.make_async_copy(ebuf.at[s], acc_v.at[idx_v.at[s]],
                              ssem[s]).wait()

    pltpu.async_copy(_chunk_src(0), ebuf.at[0], dsem[0])
    pltpu.async_copy(_chunk_src(1), ebuf.at[1], dsem[1])

    def _turn(cc, carry):
        for s in range(NBUF):
            c = cc * NBUF + s

            @pl.when(c < NCH)
            def _process():
                pltpu.make_async_copy(_chunk_src(0), ebuf.at[s],
                                      dsem[s]).wait()
                _fill_idx(c, s)
                pltpu.async_copy(ebuf.at[s], acc_v.at[idx_v.at[s]], ssem[s],
                                 add=True)
                for k in range(CHUNK // LN):
                    sv = idx_v[s, pl.ds(k * LN, LN)]
                    tgt = sv * LN + lane_iota
                    plsc.addupdate_scatter(cnt_v, [tgt], ones16)

            sp = (s + 2) % NBUF

            @pl.when(c + 2 < NCH)
            def _prefetch():
                @pl.when(c >= 2)
                def _drain_prev():
                    _scatter_drain(sp)
                pltpu.async_copy(_chunk_src(c + 2), ebuf.at[sp], dsem[sp])
        return carry
    lax.fori_loop(0, (NCH + NBUF - 1) // NBUF, _turn, 0)

    for sf in range(NBUF):
        _scatter_drain(sf)
    plsc.subcore_barrier()

    @pl.when(sid == 0)
    def _out_sums():
        pltpu.sync_copy(acc_v, sums_hbm.at[cid])
    pltpu.sync_copy(cnt_v, cnts_hbm.at[wid])


def _tc_x_body(x_ref, batch_ref, row_ref, e_ref, xm_ref, es_ref, ec_ref,
               starts_scr, hist_scr, acc_scr, cnt_scr):
    i = pl.program_id(0)

    @pl.when(i == 0)
    def _init():
        b_iota = jax.lax.broadcasted_iota(jnp.int32, (B, N), 0)
        hist_col = jnp.sum(jnp.equal(batch_ref[...], b_iota).astype(jnp.float32),
                           axis=1, keepdims=True)
        tri = (jax.lax.broadcasted_iota(jnp.int32, (B, B), 0)
               > jax.lax.broadcasted_iota(jnp.int32, (B, B), 1)).astype(jnp.float32)
        starts_col = jnp.dot(tri, hist_col, preferred_element_type=jnp.float32)
        starts_scr[...] = jnp.broadcast_to(starts_col.astype(jnp.int32), (B, H))
        hist_scr[...] = jnp.broadcast_to(hist_col.astype(jnp.int32), (B, H))
        acc_scr[...] = jnp.zeros((B, H), jnp.float32)
        cnt_scr[...] = jnp.zeros((B, H), jnp.float32)

    s_col = starts_scr[:, 0:1]
    h_col = hist_scr[:, 0:1]
    e_col = s_col + h_col

    row2 = row_ref[...].reshape(1, BK)
    mask = (row2 >= s_col) & (row2 < e_col)
    onehot = mask.astype(jnp.bfloat16)
    eblk = e_ref[...].astype(jnp.bfloat16)
    acc_scr[...] += jnp.dot(onehot, eblk, preferred_element_type=jnp.float32)
    cnt_blk = jnp.sum(mask.astype(jnp.float32), axis=1, keepdims=True)
    cnt_scr[...] += jnp.broadcast_to(cnt_blk, (B, H))

    @pl.when(i == NBK - 1)
    def _finish():
        n_iota = jax.lax.broadcasted_iota(jnp.int32, (B, N), 1)
        maskx = ((n_iota >= s_col) & (n_iota < e_col)).astype(jnp.float32)
        sum_x = jnp.dot(maskx, x_ref[...], preferred_element_type=jnp.float32)
        hist_f = h_col.astype(jnp.float32)
        xm_ref[...] = sum_x / jnp.maximum(hist_f, 1.0)
        es_ref[...] = acc_scr[...]
        ec_ref[...] = cnt_scr[...]


def _tc_x(x, batch2, row3, edge_attr):
    cmap = lambda i: (0, 0)
    return pl.pallas_call(
        _tc_x_body,
        grid=(NBK,),
        in_specs=[
            pl.BlockSpec((N, H), cmap),
            pl.BlockSpec((1, N), cmap),
            pl.BlockSpec((1, 1, BK), lambda i: (i + NBK_SC, 0, 0)),
            pl.BlockSpec((BK, H), lambda i: (i + NBK_SC, 0)),
        ],
        out_specs=[pl.BlockSpec((B, H), cmap)] * 3,
        out_shape=[jax.ShapeDtypeStruct((B, H), jnp.float32)] * 3,
        scratch_shapes=[
            pltpu.VMEM((B, H), jnp.int32),
            pltpu.VMEM((B, H), jnp.int32),
            pltpu.VMEM((B, H), jnp.float32),
            pltpu.VMEM((B, H), jnp.float32),
        ],
        compiler_params=pltpu.CompilerParams(
            dimension_semantics=("arbitrary",),
        ),
    )(x, batch2, row3, edge_attr)


def _tc_combine_body(ps_ref, pc_ref, es_ref, ec_ref, xm_ref, u_ref, w1_ref,
                     b1_ref, w2_ref, b2_ref, out_ref):
    dn = (((1,), (1,)), ((), ()))
    e_sum = jnp.sum(ps_ref[...], axis=0) + es_ref[...]
    cnt_col = (jnp.sum(jnp.sum(pc_ref[...], axis=0), axis=1,
                       keepdims=True) + ec_ref[:, 0:1])
    e_mean = e_sum / jnp.maximum(cnt_col, 1.0)
    cat = jnp.concatenate([u_ref[...], xm_ref[...], e_mean], axis=1)
    h1 = jax.lax.dot_general(cat, w1_ref[...], dn,
                             preferred_element_type=jnp.float32) + b1_ref[...]
    h1 = jnp.maximum(h1, 0.0)
    out_ref[...] = jax.lax.dot_general(h1, w2_ref[...], dn,
                                       preferred_element_type=jnp.float32) + b2_ref[...]


def _tc_combine(part_sums, part_cnts, e_tc_sum, e_tc_cnt, x_mean, u, W1, b1r,
                W2, b2r):
    return pl.pallas_call(
        _tc_combine_body,
        grid=(1,),
        in_specs=[
            pl.BlockSpec((NC, B, H), lambda i: (0, 0, 0)),
            pl.BlockSpec((NW, B, LN), lambda i: (0, 0, 0)),
            pl.BlockSpec((B, H), lambda i: (0, 0)),
            pl.BlockSpec((B, H), lambda i: (0, 0)),
            pl.BlockSpec((B, H), lambda i: (0, 0)),
            pl.BlockSpec((B, H), lambda i: (0, 0)),
            pl.BlockSpec((H, 3 * H), lambda i: (0, 0)),
            pl.BlockSpec((1, H), lambda i: (0, 0)),
            pl.BlockSpec((H, H), lambda i: (0, 0)),
            pl.BlockSpec((1, H), lambda i: (0, 0)),
        ],
        out_specs=pl.BlockSpec((B, H), lambda i: (0, 0)),
        out_shape=jax.ShapeDtypeStruct((B, H), jnp.float32),
        compiler_params=pltpu.CompilerParams(
            dimension_semantics=("arbitrary",),
        ),
    )(part_sums, part_cnts, e_tc_sum, e_tc_cnt, x_mean, u, W1, b1r, W2, b2r)


def kernel(x, edge_index, edge_attr, u, batch, W1, b1, W2, b2):
    row = edge_index[0]
    zsum = jnp.zeros((B, H), jnp.float32)
    part_sums, part_cnts = _sc_edge_pool(row, batch, edge_attr, zsum)
    x_mean, e_tc_sum, e_tc_cnt = _tc_x(x, batch.reshape(1, N),
                                       row.reshape(NBK_ALL, 1, BK), edge_attr)
    return _tc_combine(part_sums, part_cnts.reshape(NW, B, LN), e_tc_sum,
                       e_tc_cnt, x_mean, u, W1, b1.reshape(1, H), W2,
                       b2.reshape(1, H))
